# baseline (device time: 362125 ns/iter reference)
import jax
import jax.numpy as jnp
from jax import lax
from jax.experimental import pallas as pl
from jax.experimental.pallas import tpu as pltpu

N_DEV = 8
B_LOC = 2
SQ = 512
SKV = 512
HQ_LOC = 8
DH = 64
D_MODEL = 768
D_CHUNK = HQ_LOC * DH


def kernel(x, Wq, K_ext, V_ext, Wo):
    my = lax.axis_index("i")
    Kb = lax.dynamic_slice_in_dim(K_ext, my * B_LOC, B_LOC, axis=0)
    Vb = lax.dynamic_slice_in_dim(V_ext, my * B_LOC, B_LOC, axis=0)

    def body(x_ref, wq_ref, k_ref, v_ref, wo_ref, out_ref,
             wq_comm, wo_comm, k_scr, v_scr, q_scr, ctx_scr,
             wq_send, wq_recv, wo_send, wo_recv, k_sems, v_sems):
        my = lax.axis_index("i")
        left = (my + N_DEV - 1) % N_DEV
        right = (my + 1) % N_DEV

        barrier = pltpu.get_barrier_semaphore()
        for nbr in (left, right):
            pl.semaphore_signal(barrier, inc=1, device_id=(nbr,),
                                device_id_type=pl.DeviceIdType.MESH)
        pl.semaphore_wait(barrier, 2)

        wq_comm[0] = wq_ref[...]
        wo_comm[0] = wo_ref[...]

        def kv_copies(g, slot):
            copies = []
            for hh in range(HQ_LOC):
                copies.append(pltpu.make_async_copy(
                    k_ref.at[:, :, g * HQ_LOC + hh, :],
                    k_scr.at[slot, :, hh], k_sems.at[slot]))
                copies.append(pltpu.make_async_copy(
                    v_ref.at[:, :, g * HQ_LOC + hh, :],
                    v_scr.at[slot, :, hh], v_sems.at[slot]))
            return copies

        pending = kv_copies(my, 0)
        for c in pending:
            c.start()

        for h in range(N_DEV):
            slot = h % 2
            nxt = (h + 1) % 2
            g = (my + N_DEV - h) % N_DEV

            rdmas = []
            if h < N_DEV - 1:
                for comm, ssem, rsem in ((wq_comm, wq_send, wq_recv),
                                         (wo_comm, wo_send, wo_recv)):
                    r = pltpu.make_async_remote_copy(
                        src_ref=comm.at[slot], dst_ref=comm.at[nxt],
                        send_sem=ssem.at[slot], recv_sem=rsem.at[nxt],
                        device_id=(right,),
                        device_id_type=pl.DeviceIdType.MESH)
                    r.start()
                    rdmas.append(r)
                nxt_copies = kv_copies((my + N_DEV - h - 1) % N_DEV, nxt)
                for c in nxt_copies:
                    c.start()

            for c in pending:
                c.wait()

            for b in range(B_LOC):
                qfull = jnp.dot(x_ref[b], wq_comm[slot],
                                preferred_element_type=jnp.float32)
                for hh in range(HQ_LOC):
                    q_scr[b, hh] = qfull[:, hh * DH:(hh + 1) * DH]

            def attn_body(idx, _, slot=slot):
                b = idx // HQ_LOC
                hh = idx % HQ_LOC
                q = q_scr[b, hh]
                k = k_scr[slot, b, hh]
                v = v_scr[slot, b, hh]
                for g4 in range(4):
                    lo = g4 * 64
                    hi = (g4 + 4) * 64
                    qg = jnp.concatenate([q[lo:lo + 64], q[hi:hi + 64]], 0)
                    kg = jnp.concatenate([k[lo:lo + 64], k[hi:hi + 64]], 0)
                    vg = jnp.concatenate([v[lo:lo + 64], v[hi:hi + 64]], 0)
                    s = lax.dot_general(qg, kg, (((1,), (1,)), ((), ())),
                                        preferred_element_type=jnp.float32)
                    s = s * 0.125
                    m = jnp.max(s, axis=1, keepdims=True)
                    e = jnp.exp(s - m)
                    w = e / jnp.sum(e, axis=1, keepdims=True)
                    ctx = lax.dot_general(w, vg, (((1,), (0,)), ((), ())),
                                          preferred_element_type=jnp.float32)
                    ctx_scr[b, hh, lo:lo + 64] = ctx[:64]
                    ctx_scr[b, hh, hi:hi + 64] = ctx[64:]
                return 0

            lax.fori_loop(0, B_LOC * HQ_LOC, attn_body, 0)

            for b in range(B_LOC):
                ctx2d = jnp.concatenate(
                    [ctx_scr[b, hh] for hh in range(HQ_LOC)], axis=1)
                contrib = jnp.dot(ctx2d, wo_comm[slot],
                                  preferred_element_type=jnp.float32)
                if h == 0:
                    out_ref[b] = contrib
                else:
                    out_ref[b] = out_ref[b] + contrib

            if h < N_DEV - 1:
                for r in rdmas:
                    r.wait()
                pending = nxt_copies

    return pl.pallas_call(
        body,
        out_shape=jax.ShapeDtypeStruct((B_LOC, SQ, D_MODEL), jnp.float32),
        in_specs=[
            pl.BlockSpec(memory_space=pltpu.VMEM),
            pl.BlockSpec(memory_space=pltpu.VMEM),
            pl.BlockSpec(memory_space=pl.ANY),
            pl.BlockSpec(memory_space=pl.ANY),
            pl.BlockSpec(memory_space=pltpu.VMEM),
        ],
        out_specs=pl.BlockSpec(memory_space=pltpu.VMEM),
        scratch_shapes=[
            pltpu.VMEM((2, D_MODEL, D_CHUNK), jnp.float32),
            pltpu.VMEM((2, D_CHUNK, D_MODEL), jnp.float32),
            pltpu.VMEM((2, B_LOC, HQ_LOC, SKV, DH), jnp.float32),
            pltpu.VMEM((2, B_LOC, HQ_LOC, SKV, DH), jnp.float32),
            pltpu.VMEM((B_LOC, HQ_LOC, SQ, DH), jnp.float32),
            pltpu.VMEM((B_LOC, HQ_LOC, SQ, DH), jnp.float32),
            pltpu.SemaphoreType.DMA((2,)),
            pltpu.SemaphoreType.DMA((2,)),
            pltpu.SemaphoreType.DMA((2,)),
            pltpu.SemaphoreType.DMA((2,)),
            pltpu.SemaphoreType.DMA((2,)),
            pltpu.SemaphoreType.DMA((2,)),
        ],
        compiler_params=pltpu.CompilerParams(
            collective_id=0, vmem_limit_bytes=50 * 1024 * 1024),
    )(x, Wq, Kb, Vb, Wo)


# device time: 301583 ns/iter; 1.2007x vs baseline; 1.2007x over previous
import jax
import jax.numpy as jnp
from jax import lax
from jax.experimental import pallas as pl
from jax.experimental.pallas import tpu as pltpu

N_DEV = 8
B_LOC = 2
SQ = 512
SKV = 512
HQ_LOC = 8
DH = 64
D_MODEL = 768
D_CHUNK = HQ_LOC * DH


def kernel(x, Wq, K_ext, V_ext, Wo):
    my = lax.axis_index("i")
    Kb = lax.dynamic_slice_in_dim(K_ext, my * B_LOC, B_LOC, axis=0)
    Vb = lax.dynamic_slice_in_dim(V_ext, my * B_LOC, B_LOC, axis=0)

    def body(x_ref, wq_ref, k_ref, v_ref, wo_ref, out_ref,
             wq_r, wo_r, wq_l, wo_l,
             k_scr, v_scr, q_scr, ctx_scr,
             sr_wq, rr_wq, sr_wo, rr_wo, sl_wq, rl_wq, sl_wo, rl_wo,
             k_sems, v_sems):
        my = lax.axis_index("i")
        left = (my + N_DEV - 1) % N_DEV
        right = (my + 1) % N_DEV

        barrier = pltpu.get_barrier_semaphore()
        for nbr in (left, right):
            pl.semaphore_signal(barrier, inc=1, device_id=(nbr,),
                                device_id_type=pl.DeviceIdType.MESH)
        pl.semaphore_wait(barrier, 2)

        def kv_copies(g, par, d):
            copies = []
            for hh in range(HQ_LOC):
                copies.append(pltpu.make_async_copy(
                    k_ref.at[:, :, g * HQ_LOC + hh, :],
                    k_scr.at[par, d, :, hh], k_sems.at[par, d]))
                copies.append(pltpu.make_async_copy(
                    v_ref.at[:, :, g * HQ_LOC + hh, :],
                    v_scr.at[par, d, :, hh], v_sems.at[par, d]))
            for c in copies:
                c.start()
            return copies

        def send_pair(src_wq, src_wo, dst_slot, dev,
                      comm_wq, comm_wo, ss_wq, ss_wo, rs_wq, rs_wo):
            rs = []
            for src, comm, ssem, rsem in ((src_wq, comm_wq, ss_wq, rs_wq),
                                          (src_wo, comm_wo, ss_wo, rs_wo)):
                r = pltpu.make_async_remote_copy(
                    src_ref=src, dst_ref=comm.at[dst_slot],
                    send_sem=ssem.at[dst_slot], recv_sem=rsem.at[dst_slot],
                    device_id=(dev,), device_id_type=pl.DeviceIdType.MESH)
                r.start()
                rs.append(r)
            return rs

        def wait_in(comm, rsem, sl):
            pltpu.make_async_remote_copy(
                src_ref=comm.at[sl], dst_ref=comm.at[sl],
                send_sem=rsem.at[sl], recv_sem=rsem.at[sl],
                device_id=(left,),
                device_id_type=pl.DeviceIdType.MESH).wait_recv()

        def compute_chunk(wq_src, wo_src, par, d, first):
            for b in range(B_LOC):
                qfull = jnp.dot(x_ref[b], wq_src[...],
                                preferred_element_type=jnp.float32)
                for hh in range(HQ_LOC):
                    q_scr[b, hh] = qfull[:, hh * DH:(hh + 1) * DH]

            def attn_body(idx, _):
                b = idx // HQ_LOC
                hh = idx % HQ_LOC
                q = q_scr[b, hh]
                k = k_scr[par, d, b, hh]
                v = v_scr[par, d, b, hh]
                for g4 in range(4):
                    lo = g4 * 64
                    hi = (g4 + 4) * 64
                    qg = jnp.concatenate([q[lo:lo + 64], q[hi:hi + 64]], 0)
                    kg = jnp.concatenate([k[lo:lo + 64], k[hi:hi + 64]], 0)
                    vg = jnp.concatenate([v[lo:lo + 64], v[hi:hi + 64]], 0)
                    s = lax.dot_general(qg, kg, (((1,), (1,)), ((), ())),
                                        preferred_element_type=jnp.float32)
                    s = s * 0.125
                    m = jnp.max(s, axis=1, keepdims=True)
                    e = jnp.exp(s - m)
                    w = e / jnp.sum(e, axis=1, keepdims=True)
                    ctx = lax.dot_general(w, vg, (((1,), (0,)), ((), ())),
                                          preferred_element_type=jnp.float32)
                    ctx_scr[b, hh, lo:lo + 64] = ctx[:64]
                    ctx_scr[b, hh, hi:hi + 64] = ctx[64:]
                return 0

            lax.fori_loop(0, B_LOC * HQ_LOC, attn_body, 0)

            for b in range(B_LOC):
                ctx2d = jnp.concatenate(
                    [ctx_scr[b, hh] for hh in range(HQ_LOC)], axis=1)
                contrib = jnp.dot(ctx2d, wo_src[...],
                                  preferred_element_type=jnp.float32)
                if first:
                    out_ref[b] = contrib
                else:
                    out_ref[b] = out_ref[b] + contrib

        own_sends = (
            send_pair(wq_ref, wo_ref, 1, right,
                      wq_r, wo_r, sr_wq, sr_wo, rr_wq, rr_wo)
            + send_pair(wq_ref, wo_ref, 1, left,
                        wq_l, wo_l, sl_wq, sl_wo, rl_wq, rl_wo))

        pend = {}
        pend[(0, 0)] = kv_copies(my, 0, 0)
        pend[(1, 0)] = kv_copies((my + N_DEV - 1) % N_DEV, 1, 0)
        pend[(1, 1)] = kv_copies((my + 1) % N_DEV, 1, 1)

        for c in pend[(0, 0)]:
            c.wait()
        compute_chunk(wq_ref, wo_ref, 0, 0, first=True)
        for r in own_sends:
            r.wait_send()

        for s in range(1, 5):
            sl = s % 2
            nx = (s + 1) % 2

            if s < 4:
                pend[(nx, 0)] = kv_copies((my + N_DEV - (s + 1)) % N_DEV,
                                          nx, 0)
                if s + 1 <= 3:
                    pend[(nx, 1)] = kv_copies((my + s + 1) % N_DEV, nx, 1)

            wait_in(wq_r, rr_wq, sl)
            wait_in(wo_r, rr_wo, sl)
            fwd = []
            if s < 4:
                fwd += send_pair(wq_r.at[sl], wo_r.at[sl], nx, right,
                                 wq_r, wo_r, sr_wq, sr_wo, rr_wq, rr_wo)
            if s <= 3:
                wait_in(wq_l, rl_wq, sl)
                wait_in(wo_l, rl_wo, sl)
                if s < 3:
                    fwd += send_pair(wq_l.at[sl], wo_l.at[sl], nx, left,
                                     wq_l, wo_l, sl_wq, sl_wo, rl_wq, rl_wo)

            for c in pend[(sl, 0)]:
                c.wait()
            compute_chunk(wq_r.at[sl], wo_r.at[sl], sl, 0, first=False)
            if s <= 3:
                for c in pend[(sl, 1)]:
                    c.wait()
                compute_chunk(wq_l.at[sl], wo_l.at[sl], sl, 1, first=False)

            for r in fwd:
                r.wait_send()

    return pl.pallas_call(
        body,
        out_shape=jax.ShapeDtypeStruct((B_LOC, SQ, D_MODEL), jnp.float32),
        in_specs=[
            pl.BlockSpec(memory_space=pltpu.VMEM),
            pl.BlockSpec(memory_space=pltpu.VMEM),
            pl.BlockSpec(memory_space=pl.ANY),
            pl.BlockSpec(memory_space=pl.ANY),
            pl.BlockSpec(memory_space=pltpu.VMEM),
        ],
        out_specs=pl.BlockSpec(memory_space=pltpu.VMEM),
        scratch_shapes=[
            pltpu.VMEM((2, D_MODEL, D_CHUNK), jnp.float32),
            pltpu.VMEM((2, D_CHUNK, D_MODEL), jnp.float32),
            pltpu.VMEM((2, D_MODEL, D_CHUNK), jnp.float32),
            pltpu.VMEM((2, D_CHUNK, D_MODEL), jnp.float32),
            pltpu.VMEM((2, 2, B_LOC, HQ_LOC, SKV, DH), jnp.float32),
            pltpu.VMEM((2, 2, B_LOC, HQ_LOC, SKV, DH), jnp.float32),
            pltpu.VMEM((B_LOC, HQ_LOC, SQ, DH), jnp.float32),
            pltpu.VMEM((B_LOC, HQ_LOC, SQ, DH), jnp.float32),
            pltpu.SemaphoreType.DMA((2,)),
            pltpu.SemaphoreType.DMA((2,)),
            pltpu.SemaphoreType.DMA((2,)),
            pltpu.SemaphoreType.DMA((2,)),
            pltpu.SemaphoreType.DMA((2,)),
            pltpu.SemaphoreType.DMA((2,)),
            pltpu.SemaphoreType.DMA((2,)),
            pltpu.SemaphoreType.DMA((2,)),
            pltpu.SemaphoreType.DMA((2, 2)),
            pltpu.SemaphoreType.DMA((2, 2)),
        ],
        compiler_params=pltpu.CompilerParams(
            collective_id=0, vmem_limit_bytes=63 * 1024 * 1024),
    )(x, Wq, Kb, Vb, Wo)


# device time: 290236 ns/iter; 1.2477x vs baseline; 1.0391x over previous
import jax
import jax.numpy as jnp
from jax import lax
from jax.experimental import pallas as pl
from jax.experimental.pallas import tpu as pltpu

N_DEV = 8
B_LOC = 2
SQ = 512
SKV = 512
HQ_LOC = 8
DH = 64
D_MODEL = 768
D_CHUNK = HQ_LOC * DH


def kernel(x, Wq, K_ext, V_ext, Wo):
    my = lax.axis_index("i")
    Kb = lax.dynamic_slice_in_dim(K_ext, my * B_LOC, B_LOC, axis=0)
    Vb = lax.dynamic_slice_in_dim(V_ext, my * B_LOC, B_LOC, axis=0)

    def body(x_ref, wq_ref, k_ref, v_ref, wo_ref, out_ref,
             wq_own, wo_own, x_bf, wq_r, wo_r, wq_l, wo_l,
             k_scr, v_scr, q_scr, ctx_scr,
             sr_wq, rr_wq, sr_wo, rr_wo, sl_wq, rl_wq, sl_wo, rl_wo,
             k_sems, v_sems):
        my = lax.axis_index("i")
        left = (my + N_DEV - 1) % N_DEV
        right = (my + 1) % N_DEV

        barrier = pltpu.get_barrier_semaphore()
        for nbr in (left, right):
            pl.semaphore_signal(barrier, inc=1, device_id=(nbr,),
                                device_id_type=pl.DeviceIdType.MESH)
        pl.semaphore_wait(barrier, 2)

        def kv_copies(g, par, d):
            copies = []
            for hh in range(HQ_LOC):
                copies.append(pltpu.make_async_copy(
                    k_ref.at[:, :, g * HQ_LOC + hh, :],
                    k_scr.at[par, d, :, hh], k_sems.at[par, d]))
                copies.append(pltpu.make_async_copy(
                    v_ref.at[:, :, g * HQ_LOC + hh, :],
                    v_scr.at[par, d, :, hh], v_sems.at[par, d]))
            for c in copies:
                c.start()
            return copies

        def send_pair(src_wq, src_wo, dst_slot, dev,
                      comm_wq, comm_wo, ss_wq, ss_wo, rs_wq, rs_wo):
            rs = []
            for src, comm, ssem, rsem in ((src_wq, comm_wq, ss_wq, rs_wq),
                                          (src_wo, comm_wo, ss_wo, rs_wo)):
                r = pltpu.make_async_remote_copy(
                    src_ref=src, dst_ref=comm.at[dst_slot],
                    send_sem=ssem.at[dst_slot], recv_sem=rsem.at[dst_slot],
                    device_id=(dev,), device_id_type=pl.DeviceIdType.MESH)
                r.start()
                rs.append(r)
            return rs

        def wait_in(comm, rsem, sl):
            pltpu.make_async_remote_copy(
                src_ref=comm.at[sl], dst_ref=comm.at[sl],
                send_sem=rsem.at[sl], recv_sem=rsem.at[sl],
                device_id=(left,),
                device_id_type=pl.DeviceIdType.MESH).wait_recv()

        def compute_chunk(wq_src, wo_src, par, d, first):
            for b in range(B_LOC):
                qfull = jnp.dot(x_bf[b], wq_src[...],
                                preferred_element_type=jnp.float32)
                for hh in range(HQ_LOC):
                    q_scr[b, hh] = qfull[:, hh * DH:(hh + 1) * DH]

            def attn_body(idx, _):
                b = idx // HQ_LOC
                hh = idx % HQ_LOC
                q = q_scr[b, hh]
                k = k_scr[par, d, b, hh]
                v = v_scr[par, d, b, hh]
                for g4 in range(4):
                    lo = g4 * 64
                    hi = (g4 + 4) * 64
                    qg = jnp.concatenate([q[lo:lo + 64], q[hi:hi + 64]], 0)
                    kg = jnp.concatenate([k[lo:lo + 64], k[hi:hi + 64]], 0)
                    vg = jnp.concatenate([v[lo:lo + 64], v[hi:hi + 64]], 0)
                    s = lax.dot_general(qg, kg, (((1,), (1,)), ((), ())),
                                        preferred_element_type=jnp.float32)
                    s = s * 0.125
                    m = jnp.max(s, axis=1, keepdims=True)
                    e = jnp.exp(s - m)
                    w = e / jnp.sum(e, axis=1, keepdims=True)
                    ctx = lax.dot_general(w, vg, (((1,), (0,)), ((), ())),
                                          preferred_element_type=jnp.float32)
                    ctx_scr[b, hh, lo:lo + 64] = ctx[:64]
                    ctx_scr[b, hh, hi:hi + 64] = ctx[64:]
                return 0

            lax.fori_loop(0, B_LOC * HQ_LOC, attn_body, 0)

            for b in range(B_LOC):
                ctx2d = jnp.concatenate(
                    [ctx_scr[b, hh] for hh in range(HQ_LOC)], axis=1)
                contrib = jnp.dot(ctx2d.astype(jnp.bfloat16), wo_src[...],
                                  preferred_element_type=jnp.float32)
                if first:
                    out_ref[b] = contrib
                else:
                    out_ref[b] = out_ref[b] + contrib

        wq_own[...] = wq_ref[...].astype(jnp.bfloat16)
        wo_own[...] = wo_ref[...].astype(jnp.bfloat16)
        x_bf[...] = x_ref[...].astype(jnp.bfloat16)
        own_sends = (
            send_pair(wq_own, wo_own, 1, right,
                      wq_r, wo_r, sr_wq, sr_wo, rr_wq, rr_wo)
            + send_pair(wq_own, wo_own, 1, left,
                        wq_l, wo_l, sl_wq, sl_wo, rl_wq, rl_wo))

        pend = {}
        pend[(0, 0)] = kv_copies(my, 0, 0)
        pend[(1, 0)] = kv_copies((my + N_DEV - 1) % N_DEV, 1, 0)
        pend[(1, 1)] = kv_copies((my + 1) % N_DEV, 1, 1)

        for c in pend[(0, 0)]:
            c.wait()
        compute_chunk(wq_own, wo_own, 0, 0, first=True)
        for r in own_sends:
            r.wait_send()

        for s in range(1, 5):
            sl = s % 2
            nx = (s + 1) % 2

            if s < 4:
                pend[(nx, 0)] = kv_copies((my + N_DEV - (s + 1)) % N_DEV,
                                          nx, 0)
                if s + 1 <= 3:
                    pend[(nx, 1)] = kv_copies((my + s + 1) % N_DEV, nx, 1)

            wait_in(wq_r, rr_wq, sl)
            wait_in(wo_r, rr_wo, sl)
            fwd = []
            if s < 4:
                fwd += send_pair(wq_r.at[sl], wo_r.at[sl], nx, right,
                                 wq_r, wo_r, sr_wq, sr_wo, rr_wq, rr_wo)
            if s <= 3:
                wait_in(wq_l, rl_wq, sl)
                wait_in(wo_l, rl_wo, sl)
                if s < 3:
                    fwd += send_pair(wq_l.at[sl], wo_l.at[sl], nx, left,
                                     wq_l, wo_l, sl_wq, sl_wo, rl_wq, rl_wo)

            for c in pend[(sl, 0)]:
                c.wait()
            compute_chunk(wq_r.at[sl], wo_r.at[sl], sl, 0, first=False)
            if s <= 3:
                for c in pend[(sl, 1)]:
                    c.wait()
                compute_chunk(wq_l.at[sl], wo_l.at[sl], sl, 1, first=False)

            for r in fwd:
                r.wait_send()

    return pl.pallas_call(
        body,
        out_shape=jax.ShapeDtypeStruct((B_LOC, SQ, D_MODEL), jnp.float32),
        in_specs=[
            pl.BlockSpec(memory_space=pltpu.VMEM),
            pl.BlockSpec(memory_space=pltpu.VMEM),
            pl.BlockSpec(memory_space=pl.ANY),
            pl.BlockSpec(memory_space=pl.ANY),
            pl.BlockSpec(memory_space=pltpu.VMEM),
        ],
        out_specs=pl.BlockSpec(memory_space=pltpu.VMEM),
        scratch_shapes=[
            pltpu.VMEM((D_MODEL, D_CHUNK), jnp.bfloat16),
            pltpu.VMEM((D_CHUNK, D_MODEL), jnp.bfloat16),
            pltpu.VMEM((B_LOC, SQ, D_MODEL), jnp.bfloat16),
            pltpu.VMEM((2, D_MODEL, D_CHUNK), jnp.bfloat16),
            pltpu.VMEM((2, D_CHUNK, D_MODEL), jnp.bfloat16),
            pltpu.VMEM((2, D_MODEL, D_CHUNK), jnp.bfloat16),
            pltpu.VMEM((2, D_CHUNK, D_MODEL), jnp.bfloat16),
            pltpu.VMEM((2, 2, B_LOC, HQ_LOC, SKV, DH), jnp.float32),
            pltpu.VMEM((2, 2, B_LOC, HQ_LOC, SKV, DH), jnp.float32),
            pltpu.VMEM((B_LOC, HQ_LOC, SQ, DH), jnp.float32),
            pltpu.VMEM((B_LOC, HQ_LOC, SQ, DH), jnp.float32),
            pltpu.SemaphoreType.DMA((2,)),
            pltpu.SemaphoreType.DMA((2,)),
            pltpu.SemaphoreType.DMA((2,)),
            pltpu.SemaphoreType.DMA((2,)),
            pltpu.SemaphoreType.DMA((2,)),
            pltpu.SemaphoreType.DMA((2,)),
            pltpu.SemaphoreType.DMA((2,)),
            pltpu.SemaphoreType.DMA((2,)),
            pltpu.SemaphoreType.DMA((2, 2)),
            pltpu.SemaphoreType.DMA((2, 2)),
        ],
        compiler_params=pltpu.CompilerParams(
            collective_id=0, vmem_limit_bytes=63 * 1024 * 1024),
    )(x, Wq, Kb, Vb, Wo)
